# baseline (speedup 1.0000x reference)
import jax
import jax.numpy as jnp
from jax.experimental import pallas as pl
from jax.experimental.pallas import tpu as pltpu

_CHUNK = 128
_HALVES = 2


def _body(m_ref, mn_ref, x_hbm, o_ref, scratch, sems):
    g = pl.program_id(0)
    ng = pl.num_programs(0)
    _, S2, D = scratch.shape
    nc = S2 // _CHUNK

    length = jnp.sum(m_ref[0, 0, :])
    length_nxt = jnp.sum(mn_ref[0, 0, :])

    def chunk_copy(step, buf, c):
        row = step // _HALVES
        base = (step % _HALVES) * S2
        return pltpu.make_async_copy(
            x_hbm.at[row, pl.ds(base + c * _CHUNK, _CHUNK), :],
            scratch.at[buf, pl.ds(c * _CHUNK, _CHUNK), :],
            sems.at[buf],
        )

    def issue(step, buf, row_len):
        nch = (row_len - (step % _HALVES) * S2 + _CHUNK - 1) // _CHUNK

        def st(c, carry):
            @pl.when(c < nch)
            def _():
                chunk_copy(step, buf, c).start()
            return carry

        jax.lax.fori_loop(0, nc, st, 0, unroll=True)

    def wait_step(step, buf, row_len):
        nch = (row_len - (step % _HALVES) * S2 + _CHUNK - 1) // _CHUNK

        def wt(c, carry):
            @pl.when(c < nch)
            def _():
                chunk_copy(step, buf, c).wait()
            return carry

        jax.lax.fori_loop(0, nc, wt, 0, unroll=True)

    @pl.when(g == 0)
    def _():
        issue(g, 0, length)

    nxt = g + 1

    @pl.when((nxt < ng) & (nxt % 2 == 0))
    def _():
        issue(nxt, 0, length_nxt)

    @pl.when((nxt < ng) & (nxt % 2 == 1))
    def _():
        issue(nxt, 1, length_nxt)

    @pl.when(g % 2 == 0)
    def _():
        wait_step(g, 0, length)

    @pl.when(g % 2 == 1)
    def _():
        wait_step(g, 1, length)

    base = (g % _HALVES) * S2
    zeros_c = jnp.zeros((_CHUNK, D), dtype=o_ref.dtype)
    for buf in (0, 1):

        @pl.when(g % 2 == buf)
        def _(buf=buf):
            for c in range(nc):
                lo = c * _CHUNK

                @pl.when(base + lo < length)
                def _(lo=lo):
                    pos = (
                        jax.lax.broadcasted_iota(jnp.int32, (_CHUNK, 1), 0)
                        + base
                        + lo
                    )
                    m_t = m_ref[0, 0, pl.ds(base + lo, _CHUNK)][:, None]
                    keep = (pos < length) & (m_t > 0)
                    o_ref[0, 0, pl.ds(lo, _CHUNK), :] = jnp.where(
                        keep, scratch[buf, pl.ds(lo, _CHUNK), :], zeros_c
                    )

                @pl.when(base + lo >= length)
                def _(lo=lo):
                    o_ref[0, 0, pl.ds(lo, _CHUNK), :] = zeros_c


def kernel(x, mask):
    B, S, D = x.shape
    S2 = S // _HALVES
    m3 = mask.astype(jnp.int32).reshape(B, 1, S)
    ng = B * _HALVES
    out = pl.pallas_call(
        _body,
        grid=(ng,),
        in_specs=[
            pl.BlockSpec((1, 1, S), lambda g: (g // _HALVES, 0, 0)),
            pl.BlockSpec(
                (1, 1, S), lambda g: (jnp.minimum(g + 1, ng - 1) // _HALVES, 0, 0)
            ),
            pl.BlockSpec(memory_space=pl.ANY),
        ],
        out_specs=pl.BlockSpec(
            (1, 1, S2, D), lambda g: (g // _HALVES, g % _HALVES, 0, 0)
        ),
        out_shape=jax.ShapeDtypeStruct((B, _HALVES, S2, D), x.dtype),
        scratch_shapes=[
            pltpu.VMEM((2, S2, D), x.dtype),
            pltpu.SemaphoreType.DMA((2,)),
        ],
    )(m3, m3, x)
    return out.reshape(B, S, D)


# speedup vs baseline: 1.0010x; 1.0010x over previous
import jax
import jax.numpy as jnp
from jax.experimental import pallas as pl
from jax.experimental.pallas import tpu as pltpu

_CHUNK = 128
_HALVES = 2


def _body(m_ref, mn_ref, x_hbm, o_ref, scratch, sems):
    g = pl.program_id(0)
    ng = pl.num_programs(0)
    _, S2, D = scratch.shape
    nc = S2 // _CHUNK

    length = jnp.sum(m_ref[0, 0, :])
    length_nxt = jnp.sum(mn_ref[0, 0, :])

    def chunk_copy(step, buf, c):
        row = step // _HALVES
        base = (step % _HALVES) * S2
        return pltpu.make_async_copy(
            x_hbm.at[row, pl.ds(base + c * _CHUNK, _CHUNK), :],
            scratch.at[buf, pl.ds(c * _CHUNK, _CHUNK), :],
            sems.at[buf],
        )

    def issue(step, buf, row_len):
        nch = (row_len - (step % _HALVES) * S2 + _CHUNK - 1) // _CHUNK

        def st(c, carry):
            @pl.when(c < nch)
            def _():
                chunk_copy(step, buf, c).start()
            return carry

        jax.lax.fori_loop(0, nc, st, 0, unroll=True)

    def wait_step(step, buf, row_len):
        nch = (row_len - (step % _HALVES) * S2 + _CHUNK - 1) // _CHUNK

        def wt(c, carry):
            @pl.when(c < nch)
            def _():
                chunk_copy(step, buf, c).wait()
            return carry

        jax.lax.fori_loop(0, nc, wt, 0, unroll=True)

    @pl.when(g == 0)
    def _():
        issue(g, 0, length)

    nxt = g + 1

    @pl.when((nxt < ng) & (nxt % 2 == 0))
    def _():
        issue(nxt, 0, length_nxt)

    @pl.when((nxt < ng) & (nxt % 2 == 1))
    def _():
        issue(nxt, 1, length_nxt)

    @pl.when(g % 2 == 0)
    def _():
        wait_step(g, 0, length)

    @pl.when(g % 2 == 1)
    def _():
        wait_step(g, 1, length)

    base = (g % _HALVES) * S2
    zeros_c = jnp.zeros((_CHUNK, D), dtype=o_ref.dtype)
    for buf in (0, 1):

        @pl.when(g % 2 == buf)
        def _(buf=buf):
            for c in range(nc):
                lo = c * _CHUNK

                @pl.when(base + lo < length)
                def _(lo=lo):
                    pos = (
                        jax.lax.broadcasted_iota(jnp.int32, (_CHUNK, 1), 0)
                        + base
                        + lo
                    )
                    m_t = m_ref[0, 0, pl.ds(base + lo, _CHUNK)][:, None]
                    keep = (pos < length) & (m_t > 0)
                    o_ref[0, pl.ds(lo, _CHUNK), :] = jnp.where(
                        keep, scratch[buf, pl.ds(lo, _CHUNK), :], zeros_c
                    )

                @pl.when(base + lo >= length)
                def _(lo=lo):
                    o_ref[0, pl.ds(lo, _CHUNK), :] = zeros_c


def kernel(x, mask):
    B, S, D = x.shape
    S2 = S // _HALVES
    m3 = mask.astype(jnp.int32).reshape(B, 1, S)
    ng = B * _HALVES
    return pl.pallas_call(
        _body,
        grid=(ng,),
        in_specs=[
            pl.BlockSpec((1, 1, S), lambda g: (g // _HALVES, 0, 0)),
            pl.BlockSpec(
                (1, 1, S), lambda g: (jnp.minimum(g + 1, ng - 1) // _HALVES, 0, 0)
            ),
            pl.BlockSpec(memory_space=pl.ANY),
        ],
        out_specs=pl.BlockSpec((1, S2, D), lambda g: (g // _HALVES, g % _HALVES, 0)),
        out_shape=jax.ShapeDtypeStruct((B, S, D), x.dtype),
        scratch_shapes=[
            pltpu.VMEM((2, S2, D), x.dtype),
            pltpu.SemaphoreType.DMA((2,)),
        ],
    )(m3, m3, x)


# speedup vs baseline: 1.3826x; 1.3812x over previous
import jax
import jax.numpy as jnp
from jax.experimental import pallas as pl
from jax.experimental.pallas import tpu as pltpu

_CHUNK = 128


def _body(m_ref, mn_ref, x_hbm, o_ref, scratch, sems):
    b = pl.program_id(0)
    nb = pl.num_programs(0)
    _, S, D = scratch.shape
    nc = S // _CHUNK

    length = jnp.sum(m_ref[0, 0, :])
    length_nxt = jnp.sum(mn_ref[0, 0, :])

    def chunk_copy(row, buf, c):
        return pltpu.make_async_copy(
            x_hbm.at[row, pl.ds(c * _CHUNK, _CHUNK), :],
            scratch.at[buf, pl.ds(c * _CHUNK, _CHUNK), :],
            sems.at[buf],
        )

    def issue(row, buf, row_len):
        nch = (row_len + _CHUNK - 1) // _CHUNK

        def st(c, carry):
            @pl.when(c < nch)
            def _():
                chunk_copy(row, buf, c).start()
            return carry

        jax.lax.fori_loop(0, nc, st, 0, unroll=True)

    def wait_row(row, buf, row_len):
        nch = (row_len + _CHUNK - 1) // _CHUNK

        def wt(c, carry):
            @pl.when(c < nch)
            def _():
                chunk_copy(row, buf, c).wait()
            return carry

        jax.lax.fori_loop(0, nc, wt, 0, unroll=True)

    @pl.when(b == 0)
    def _():
        issue(b, 0, length)

    nxt = b + 1

    @pl.when((nxt < nb) & (nxt % 2 == 0))
    def _():
        issue(nxt, 0, length_nxt)

    @pl.when((nxt < nb) & (nxt % 2 == 1))
    def _():
        issue(nxt, 1, length_nxt)

    @pl.when(b % 2 == 0)
    def _():
        wait_row(b, 0, length)

    @pl.when(b % 2 == 1)
    def _():
        wait_row(b, 1, length)

    zeros_c = jnp.zeros((_CHUNK, D), dtype=o_ref.dtype)
    for buf in (0, 1):

        @pl.when(b % 2 == buf)
        def _(buf=buf):
            for c in range(nc):
                lo = c * _CHUNK

                @pl.when(lo < length)
                def _(lo=lo):
                    pos = jax.lax.broadcasted_iota(jnp.int32, (_CHUNK, 1), 0) + lo
                    m_t = m_ref[0, 0, pl.ds(lo, _CHUNK)][:, None]
                    keep = (pos < length) & (m_t > 0)
                    o_ref[0, pl.ds(lo, _CHUNK), :] = jnp.where(
                        keep, scratch[buf, pl.ds(lo, _CHUNK), :], zeros_c
                    )

                @pl.when(lo >= length)
                def _(lo=lo):
                    o_ref[0, pl.ds(lo, _CHUNK), :] = zeros_c


def kernel(x, mask):
    B, S, D = x.shape
    m3 = mask.astype(jnp.int32).reshape(B, 1, S)
    return pl.pallas_call(
        _body,
        grid=(B,),
        in_specs=[
            pl.BlockSpec((1, 1, S), lambda b: (b, 0, 0)),
            pl.BlockSpec((1, 1, S), lambda b: (jnp.minimum(b + 1, B - 1), 0, 0)),
            pl.BlockSpec(memory_space=pl.ANY),
        ],
        out_specs=pl.BlockSpec((1, S, D), lambda b: (b, 0, 0)),
        out_shape=jax.ShapeDtypeStruct((B, S, D), x.dtype),
        scratch_shapes=[
            pltpu.VMEM((2, S, D), x.dtype),
            pltpu.SemaphoreType.DMA((2,)),
        ],
    )(m3, m3, x)


# speedup vs baseline: 1.4210x; 1.0278x over previous
import jax
import jax.numpy as jnp
from jax.experimental import pallas as pl
from jax.experimental.pallas import tpu as pltpu

_CHUNK = 128
_NBUF = 3


def _body(m_ref, mn_ref, mnn_ref, x_hbm, o_ref, scratch, sems):
    b = pl.program_id(0)
    nb = pl.num_programs(0)
    _, S, D = scratch.shape
    nc = S // _CHUNK

    length = jnp.sum(m_ref[0, 0, :])
    length_n = jnp.sum(mn_ref[0, 0, :])
    length_nn = jnp.sum(mnn_ref[0, 0, :])

    def chunk_copy(row, buf, c):
        return pltpu.make_async_copy(
            x_hbm.at[row, pl.ds(c * _CHUNK, _CHUNK), :],
            scratch.at[buf, pl.ds(c * _CHUNK, _CHUNK), :],
            sems.at[buf],
        )

    def issue(row, buf, row_len):
        nch = (row_len + _CHUNK - 1) // _CHUNK

        def st(c, carry):
            @pl.when(c < nch)
            def _():
                chunk_copy(row, buf, c).start()
            return carry

        jax.lax.fori_loop(0, nc, st, 0, unroll=True)

    def wait_row(row, buf, row_len):
        nch = (row_len + _CHUNK - 1) // _CHUNK

        def wt(c, carry):
            @pl.when(c < nch)
            def _():
                chunk_copy(row, buf, c).wait()
            return carry

        jax.lax.fori_loop(0, nc, wt, 0, unroll=True)

    @pl.when(b == 0)
    def _():
        issue(b, 0, length)
        issue(b + 1, 1, length_n)

    nxt2 = b + 2
    for k in range(_NBUF):

        @pl.when((nxt2 < nb) & (nxt2 % _NBUF == k))
        def _(k=k):
            issue(nxt2, k, length_nn)

    for k in range(_NBUF):

        @pl.when(b % _NBUF == k)
        def _(k=k):
            wait_row(b, k, length)

    zeros_c = jnp.zeros((_CHUNK, D), dtype=o_ref.dtype)
    for buf in range(_NBUF):

        @pl.when(b % _NBUF == buf)
        def _(buf=buf):
            for c in range(nc):
                lo = c * _CHUNK

                @pl.when(lo < length)
                def _(lo=lo):
                    pos = jax.lax.broadcasted_iota(jnp.int32, (_CHUNK, 1), 0) + lo
                    m_t = m_ref[0, 0, pl.ds(lo, _CHUNK)][:, None]
                    keep = (pos < length) & (m_t > 0)
                    o_ref[0, pl.ds(lo, _CHUNK), :] = jnp.where(
                        keep, scratch[buf, pl.ds(lo, _CHUNK), :], zeros_c
                    )

                @pl.when(lo >= length)
                def _(lo=lo):
                    o_ref[0, pl.ds(lo, _CHUNK), :] = zeros_c


def kernel(x, mask):
    B, S, D = x.shape
    m3 = mask.astype(jnp.int32).reshape(B, 1, S)
    return pl.pallas_call(
        _body,
        grid=(B,),
        in_specs=[
            pl.BlockSpec((1, 1, S), lambda b: (b, 0, 0)),
            pl.BlockSpec((1, 1, S), lambda b: (jnp.minimum(b + 1, B - 1), 0, 0)),
            pl.BlockSpec((1, 1, S), lambda b: (jnp.minimum(b + 2, B - 1), 0, 0)),
            pl.BlockSpec(memory_space=pl.ANY),
        ],
        out_specs=pl.BlockSpec((1, S, D), lambda b: (b, 0, 0)),
        out_shape=jax.ShapeDtypeStruct((B, S, D), x.dtype),
        scratch_shapes=[
            pltpu.VMEM((_NBUF, S, D), x.dtype),
            pltpu.SemaphoreType.DMA((_NBUF,)),
        ],
    )(m3, m3, m3, x)


# speedup vs baseline: 1.4316x; 1.0074x over previous
import jax
import jax.numpy as jnp
from jax.experimental import pallas as pl
from jax.experimental.pallas import tpu as pltpu

_CHUNK = 128
_NBUF = 4


def _body(m_ref, mn_ref, mnn_ref, mnnn_ref, x_hbm, o_ref, scratch, sems):
    b = pl.program_id(0)
    nb = pl.num_programs(0)
    _, S, D = scratch.shape
    nc = S // _CHUNK

    length = jnp.sum(m_ref[0, 0, :])
    length_n = jnp.sum(mn_ref[0, 0, :])
    length_nn = jnp.sum(mnn_ref[0, 0, :])
    length_nnn = jnp.sum(mnnn_ref[0, 0, :])

    def chunk_copy(row, buf, c):
        return pltpu.make_async_copy(
            x_hbm.at[row, pl.ds(c * _CHUNK, _CHUNK), :],
            scratch.at[buf, pl.ds(c * _CHUNK, _CHUNK), :],
            sems.at[buf],
        )

    def issue(row, buf, row_len):
        nch = (row_len + _CHUNK - 1) // _CHUNK

        def st(c, carry):
            @pl.when(c < nch)
            def _():
                chunk_copy(row, buf, c).start()
            return carry

        jax.lax.fori_loop(0, nc, st, 0, unroll=True)

    def wait_row(row, buf, row_len):
        nch = (row_len + _CHUNK - 1) // _CHUNK

        def wt(c, carry):
            @pl.when(c < nch)
            def _():
                chunk_copy(row, buf, c).wait()
            return carry

        jax.lax.fori_loop(0, nc, wt, 0, unroll=True)

    @pl.when(b == 0)
    def _():
        issue(b, 0, length)
        issue(b + 1, 1, length_n)
        issue(b + 2, 2, length_nn)

    nxt3 = b + 3
    for k in range(_NBUF):

        @pl.when((nxt3 < nb) & (nxt3 % _NBUF == k))
        def _(k=k):
            issue(nxt3, k, length_nnn)

    for k in range(_NBUF):

        @pl.when(b % _NBUF == k)
        def _(k=k):
            wait_row(b, k, length)

    zeros_c = jnp.zeros((_CHUNK, D), dtype=o_ref.dtype)
    for buf in range(_NBUF):

        @pl.when(b % _NBUF == buf)
        def _(buf=buf):
            for c in range(nc):
                lo = c * _CHUNK

                @pl.when(lo < length)
                def _(lo=lo):
                    pos = jax.lax.broadcasted_iota(jnp.int32, (_CHUNK, 1), 0) + lo
                    m_t = m_ref[0, 0, pl.ds(lo, _CHUNK)][:, None]
                    keep = (pos < length) & (m_t > 0)
                    o_ref[0, pl.ds(lo, _CHUNK), :] = jnp.where(
                        keep, scratch[buf, pl.ds(lo, _CHUNK), :], zeros_c
                    )

                @pl.when(lo >= length)
                def _(lo=lo):
                    o_ref[0, pl.ds(lo, _CHUNK), :] = zeros_c


def kernel(x, mask):
    B, S, D = x.shape
    m3 = mask.astype(jnp.int32).reshape(B, 1, S)
    return pl.pallas_call(
        _body,
        grid=(B,),
        in_specs=[
            pl.BlockSpec((1, 1, S), lambda b: (b, 0, 0)),
            pl.BlockSpec((1, 1, S), lambda b: (jnp.minimum(b + 1, B - 1), 0, 0)),
            pl.BlockSpec((1, 1, S), lambda b: (jnp.minimum(b + 2, B - 1), 0, 0)),
            pl.BlockSpec((1, 1, S), lambda b: (jnp.minimum(b + 3, B - 1), 0, 0)),
            pl.BlockSpec(memory_space=pl.ANY),
        ],
        out_specs=pl.BlockSpec((1, S, D), lambda b: (b, 0, 0)),
        out_shape=jax.ShapeDtypeStruct((B, S, D), x.dtype),
        scratch_shapes=[
            pltpu.VMEM((_NBUF, S, D), x.dtype),
            pltpu.SemaphoreType.DMA((_NBUF,)),
        ],
    )(m3, m3, m3, m3, x)


# speedup vs baseline: 1.4353x; 1.0026x over previous
import jax
import jax.numpy as jnp
from jax.experimental import pallas as pl
from jax.experimental.pallas import tpu as pltpu

_CHUNK = 128
_NBUF = 5


def _body(m_ref, mn_ref, mnn_ref, mnnn_ref, m4_ref, x_hbm, o_ref, scratch, sems):
    b = pl.program_id(0)
    nb = pl.num_programs(0)
    _, S, D = scratch.shape
    nc = S // _CHUNK

    length = jnp.sum(m_ref[0, 0, :])
    length_n = jnp.sum(mn_ref[0, 0, :])
    length_nn = jnp.sum(mnn_ref[0, 0, :])
    length_nnn = jnp.sum(mnnn_ref[0, 0, :])
    length_n4 = jnp.sum(m4_ref[0, 0, :])

    def chunk_copy(row, buf, c):
        return pltpu.make_async_copy(
            x_hbm.at[row, pl.ds(c * _CHUNK, _CHUNK), :],
            scratch.at[buf, pl.ds(c * _CHUNK, _CHUNK), :],
            sems.at[buf],
        )

    def issue(row, buf, row_len):
        nch = (row_len + _CHUNK - 1) // _CHUNK

        def st(c, carry):
            @pl.when(c < nch)
            def _():
                chunk_copy(row, buf, c).start()
            return carry

        jax.lax.fori_loop(0, nc, st, 0, unroll=True)

    def wait_row(row, buf, row_len):
        nch = (row_len + _CHUNK - 1) // _CHUNK

        def wt(c, carry):
            @pl.when(c < nch)
            def _():
                chunk_copy(row, buf, c).wait()
            return carry

        jax.lax.fori_loop(0, nc, wt, 0, unroll=True)

    @pl.when(b == 0)
    def _():
        issue(b, 0, length)
        issue(b + 1, 1, length_n)
        issue(b + 2, 2, length_nn)
        issue(b + 3, 3, length_nnn)

    nxt4 = b + 4
    for k in range(_NBUF):

        @pl.when((nxt4 < nb) & (nxt4 % _NBUF == k))
        def _(k=k):
            issue(nxt4, k, length_n4)

    for k in range(_NBUF):

        @pl.when(b % _NBUF == k)
        def _(k=k):
            wait_row(b, k, length)

    zeros_c = jnp.zeros((_CHUNK, D), dtype=o_ref.dtype)
    for buf in range(_NBUF):

        @pl.when(b % _NBUF == buf)
        def _(buf=buf):
            for c in range(nc):
                lo = c * _CHUNK

                @pl.when(lo < length)
                def _(lo=lo):
                    pos = jax.lax.broadcasted_iota(jnp.int32, (_CHUNK, 1), 0) + lo
                    m_t = m_ref[0, 0, pl.ds(lo, _CHUNK)][:, None]
                    keep = (pos < length) & (m_t > 0)
                    o_ref[0, pl.ds(lo, _CHUNK), :] = jnp.where(
                        keep, scratch[buf, pl.ds(lo, _CHUNK), :], zeros_c
                    )

                @pl.when(lo >= length)
                def _(lo=lo):
                    o_ref[0, pl.ds(lo, _CHUNK), :] = zeros_c


def kernel(x, mask):
    B, S, D = x.shape
    m3 = mask.astype(jnp.int32).reshape(B, 1, S)
    return pl.pallas_call(
        _body,
        grid=(B,),
        in_specs=[
            pl.BlockSpec((1, 1, S), lambda b: (b, 0, 0)),
            pl.BlockSpec((1, 1, S), lambda b: (jnp.minimum(b + 1, B - 1), 0, 0)),
            pl.BlockSpec((1, 1, S), lambda b: (jnp.minimum(b + 2, B - 1), 0, 0)),
            pl.BlockSpec((1, 1, S), lambda b: (jnp.minimum(b + 3, B - 1), 0, 0)),
            pl.BlockSpec((1, 1, S), lambda b: (jnp.minimum(b + 4, B - 1), 0, 0)),
            pl.BlockSpec(memory_space=pl.ANY),
        ],
        out_specs=pl.BlockSpec((1, S, D), lambda b: (b, 0, 0)),
        out_shape=jax.ShapeDtypeStruct((B, S, D), x.dtype),
        scratch_shapes=[
            pltpu.VMEM((_NBUF, S, D), x.dtype),
            pltpu.SemaphoreType.DMA((_NBUF,)),
        ],
    )(m3, m3, m3, m3, m3, x)
